# Initial kernel scaffold; baseline (speedup 1.0000x reference)
import jax
import jax.numpy as jnp
from jax.experimental import pallas as pl


def kernel(x, edge_index, batch, W1, b1, W2, b2, Wlin, blin):
    raise NotImplementedError("write your pallas kernel here")


# speedup vs baseline: 19.9162x; 19.9162x over previous
import jax
import jax.numpy as jnp
from jax import lax
from jax.experimental import pallas as pl
from jax.experimental.pallas import tpu as pltpu
from jax.experimental.pallas import tpu_sc as plsc

N = 10000
E = 320000
F = 128
H = 128
C = 10
G = 128

NC = 2
NS = 16
NW = NC * NS
EP = E // NW
CH = 80
NCH = EP // CH

_SC_MESH = plsc.VectorSubcoreMesh(
    core_axis_name="c", subcore_axis_name="s", num_cores=NC, num_subcores=NS)


def _split_copy(src_ref, dst_ref, s):
    @pl.when(s < NS - 1)
    def _():
        pltpu.sync_copy(src_ref.at[pl.ds(s * 624, 624)],
                        dst_ref.at[pl.ds(s * 624, 624)])

    @pl.when(s == NS - 1)
    def _():
        pltpu.sync_copy(src_ref.at[pl.ds(9360, 640)],
                        dst_ref.at[pl.ds(9360, 640)])


def _sc_deg_body(dst_hbm, ones_hbm, zeros_hbm, out_hbm, dst_v, ones_v, acc):
    c = lax.axis_index("c")
    s = lax.axis_index("s")
    w = c * NS + s
    pltpu.sync_copy(dst_hbm.at[w], dst_v)
    pltpu.sync_copy(ones_hbm, ones_v)

    @pl.when(s == 0)
    def _():
        pltpu.sync_copy(zeros_hbm, acc)
    plsc.subcore_barrier()

    def step(j, carry):
        pltpu.sync_copy(ones_v, acc.at[dst_v.at[j]], add=True)
        return carry

    lax.fori_loop(0, NCH, step, 0)
    plsc.subcore_barrier()

    @pl.when(s == 0)
    def _():
        pltpu.sync_copy(acc, out_hbm.at[c])


_sc_deg = pl.kernel(
    _sc_deg_body,
    out_type=jax.ShapeDtypeStruct((NC, N), jnp.float32),
    mesh=_SC_MESH,
    scratch_types=[
        pltpu.VMEM((NCH, CH), jnp.int32),
        pltpu.VMEM((CH,), jnp.float32),
        pltpu.VMEM_SHARED((N,), jnp.float32),
    ],
)


def _sc_edge_body(xs_hbm, src_hbm, dst_hbm, out_hbm, src_v, dst_v, rows_v, acc):
    c = lax.axis_index("c")
    s = lax.axis_index("s")
    w = c * NS + s
    pltpu.sync_copy(src_hbm.at[w], src_v)
    pltpu.sync_copy(dst_hbm.at[w], dst_v)
    _split_copy(xs_hbm, acc, s)
    plsc.subcore_barrier()

    def step(j, carry):
        pltpu.sync_copy(xs_hbm.at[src_v.at[j]], rows_v)
        pltpu.sync_copy(rows_v, acc.at[dst_v.at[j]], add=True)
        return carry

    lax.fori_loop(0, NCH, step, 0)
    plsc.subcore_barrier()
    _split_copy(acc, out_hbm.at[c], s)


_sc_edge = pl.kernel(
    _sc_edge_body,
    out_type=jax.ShapeDtypeStruct((NC, N, H), jnp.float32),
    mesh=_SC_MESH,
    scratch_types=[
        pltpu.VMEM((NCH, CH), jnp.int32),
        pltpu.VMEM((NCH, CH), jnp.int32),
        pltpu.VMEM((CH, H), jnp.float32),
        pltpu.VMEM_SHARED((N, H), jnp.float32),
    ],
)


def _tc1_body(x_ref, w1_ref, degp_ref, xs_ref, dinv_ref):
    dp = degp_ref[...]
    dinv = lax.rsqrt(dp[0] + dp[1] + 1.0)
    xw = jnp.dot(x_ref[...], w1_ref[...], preferred_element_type=jnp.float32)
    xs_ref[...] = xw * dinv
    dinv_ref[...] = dinv


def _tc2_body(p_ref, xs1_ref, dinv_ref, b1_ref, w2_ref, xs2_ref):
    p = p_ref[...]
    dinv = dinv_ref[...]
    h1 = jnp.maximum(dinv * (p[0] + p[1] - xs1_ref[...]) + b1_ref[...], 0.0)
    xs2_ref[...] = dinv * jnp.dot(h1, w2_ref[...],
                                  preferred_element_type=jnp.float32)


def _tc3_body(q_ref, xs2_ref, dinv_ref, b2_ref, batch_ref, wlin_ref, blin_ref,
              out_ref):
    q = q_ref[...]
    dinv = dinv_ref[...]
    h2 = jnp.maximum(dinv * (q[0] + q[1] - xs2_ref[...]) + b2_ref[...], 0.0)
    gids = lax.broadcasted_iota(jnp.int32, (N, G), 1)
    onehot = (batch_ref[...] == gids).astype(jnp.float32)
    dn = (((0,), (0,)), ((), ()))
    psum = lax.dot_general(onehot, h2, dn,
                           preferred_element_type=jnp.float32)
    cnt = lax.dot_general(onehot, jnp.ones((N, 1), jnp.float32), dn,
                          preferred_element_type=jnp.float32)
    pooled = psum / jnp.maximum(cnt, 1.0)
    out_ref[...] = jnp.dot(pooled, wlin_ref[...],
                           preferred_element_type=jnp.float32) + blin_ref[...]


def kernel(x, edge_index, batch, W1, b1, W2, b2, Wlin, blin):
    src3 = edge_index[0].reshape(NW, NCH, CH)
    dst3 = edge_index[1].reshape(NW, NCH, CH)
    ones_ch = jnp.ones((CH,), jnp.float32)
    zeros_n1 = jnp.zeros((N,), jnp.float32)
    batch2d = batch.reshape(N, 1)
    b1r = b1.reshape(1, H)
    b2r = b2.reshape(1, H)
    blinr = blin.reshape(1, C)

    degp = _sc_deg(dst3, ones_ch, zeros_n1).reshape(NC, N, 1)

    xs1, dinv = pl.pallas_call(
        _tc1_body,
        out_shape=[jax.ShapeDtypeStruct((N, H), jnp.float32),
                   jax.ShapeDtypeStruct((N, 1), jnp.float32)],
    )(x, W1, degp)

    p = _sc_edge(xs1, src3, dst3)

    xs2 = pl.pallas_call(
        _tc2_body,
        out_shape=jax.ShapeDtypeStruct((N, H), jnp.float32),
    )(p, xs1, dinv, b1r, W2)

    q = _sc_edge(xs2, src3, dst3)

    logits = pl.pallas_call(
        _tc3_body,
        out_shape=jax.ShapeDtypeStruct((G, C), jnp.float32),
    )(q, xs2, dinv, b2r, batch2d, Wlin, blinr)
    return logits


# speedup vs baseline: 28.8787x; 1.4500x over previous
import jax
import jax.numpy as jnp
from jax import lax
from jax.experimental import pallas as pl
from jax.experimental.pallas import tpu as pltpu
from jax.experimental.pallas import tpu_sc as plsc

N = 10000
E = 320000
F = 128
H = 128
C = 10
G = 128

NC = 2
NS = 16
NW = NC * NS
EP = E // NW
CH = 80
NCH = EP // CH

_SC_MESH = plsc.VectorSubcoreMesh(
    core_axis_name="c", subcore_axis_name="s", num_cores=NC, num_subcores=NS)


def _split_copy(src_ref, dst_ref, s):
    @pl.when(s < NS - 1)
    def _():
        pltpu.sync_copy(src_ref.at[pl.ds(s * 624, 624)],
                        dst_ref.at[pl.ds(s * 624, 624)])

    @pl.when(s == NS - 1)
    def _():
        pltpu.sync_copy(src_ref.at[pl.ds(9360, 640)],
                        dst_ref.at[pl.ds(9360, 640)])


def _sc_deg_body(dst_hbm, ones_hbm, zeros_hbm, out_hbm, dst_v, ones_v, acc):
    c = lax.axis_index("c")
    s = lax.axis_index("s")
    w = c * NS + s
    pltpu.sync_copy(dst_hbm.at[w], dst_v)
    pltpu.sync_copy(ones_hbm, ones_v)

    @pl.when(s == 0)
    def _():
        pltpu.sync_copy(zeros_hbm, acc)
    plsc.subcore_barrier()

    def step(j, carry):
        pltpu.sync_copy(ones_v, acc.at[dst_v.at[j]], add=True)
        return carry

    lax.fori_loop(0, NCH, step, 0)
    plsc.subcore_barrier()

    @pl.when(s == 0)
    def _():
        pltpu.sync_copy(acc, out_hbm.at[c])


_sc_deg = pl.kernel(
    _sc_deg_body,
    out_type=jax.ShapeDtypeStruct((NC, N), jnp.float32),
    mesh=_SC_MESH,
    scratch_types=[
        pltpu.VMEM((NCH, CH), jnp.int32),
        pltpu.VMEM((CH,), jnp.float32),
        pltpu.VMEM_SHARED((N,), jnp.float32),
    ],
)


NPH = 5
PC = NCH // NPH


def _sc_edge_body(xs_hbm, src_hbm, dst_hbm, out_hbm,
                  src_v, dst_v, rows0, rows1, acc, sem0, sem1):
    c = lax.axis_index("c")
    s = lax.axis_index("s")
    w = c * NS + s
    _split_copy(xs_hbm, acc, s)
    plsc.subcore_barrier()

    def phase(p, carry):
        pltpu.sync_copy(src_hbm.at[w, p], src_v)
        pltpu.sync_copy(dst_hbm.at[w, p], dst_v)
        pltpu.async_copy(xs_hbm.at[src_v.at[0]], rows0, sem0)

        def step(k, carry2):
            j0 = 2 * k
            j1 = j0 + 1
            pltpu.async_copy(xs_hbm.at[src_v.at[j1]], rows1, sem1)
            pltpu.make_async_copy(xs_hbm.at[src_v.at[j0]], rows0, sem0).wait()
            pltpu.sync_copy(rows0, acc.at[dst_v.at[j0]], add=True)
            pltpu.async_copy(xs_hbm.at[src_v.at[j0 + 2]], rows0, sem0)
            pltpu.make_async_copy(xs_hbm.at[src_v.at[j1]], rows1, sem1).wait()
            pltpu.sync_copy(rows1, acc.at[dst_v.at[j1]], add=True)
            return carry2

        lax.fori_loop(0, (PC - 1) // 2, step, 0)
        pltpu.make_async_copy(xs_hbm.at[src_v.at[PC - 1]], rows0, sem0).wait()
        pltpu.sync_copy(rows0, acc.at[dst_v.at[PC - 1]], add=True)
        return carry

    lax.fori_loop(0, NPH, phase, 0)
    plsc.subcore_barrier()
    _split_copy(acc, out_hbm.at[c], s)


_sc_edge = pl.kernel(
    _sc_edge_body,
    out_type=jax.ShapeDtypeStruct((NC, N, H), jnp.float32),
    mesh=_SC_MESH,
    scratch_types=[
        pltpu.VMEM((PC, CH), jnp.int32),
        pltpu.VMEM((PC, CH), jnp.int32),
        pltpu.VMEM((CH, H), jnp.float32),
        pltpu.VMEM((CH, H), jnp.float32),
        pltpu.VMEM_SHARED((N, H), jnp.float32),
        pltpu.SemaphoreType.DMA,
        pltpu.SemaphoreType.DMA,
    ],
)


def _tc1_body(x_ref, w1_ref, degp_ref, xs_ref, dinv_ref):
    dp = degp_ref[...]
    dinv = lax.rsqrt(dp[0] + dp[1] + 1.0)
    xw = jnp.dot(x_ref[...], w1_ref[...], preferred_element_type=jnp.float32)
    xs_ref[...] = xw * dinv
    dinv_ref[...] = dinv


def _tc2_body(p_ref, xs1_ref, dinv_ref, b1_ref, w2_ref, xs2_ref):
    p = p_ref[...]
    dinv = dinv_ref[...]
    h1 = jnp.maximum(dinv * (p[0] + p[1] - xs1_ref[...]) + b1_ref[...], 0.0)
    xs2_ref[...] = dinv * jnp.dot(h1, w2_ref[...],
                                  preferred_element_type=jnp.float32)


def _tc3_body(q_ref, xs2_ref, dinv_ref, b2_ref, batch_ref, wlin_ref, blin_ref,
              out_ref):
    q = q_ref[...]
    dinv = dinv_ref[...]
    h2 = jnp.maximum(dinv * (q[0] + q[1] - xs2_ref[...]) + b2_ref[...], 0.0)
    gids = lax.broadcasted_iota(jnp.int32, (N, G), 1)
    onehot = (batch_ref[...] == gids).astype(jnp.float32)
    dn = (((0,), (0,)), ((), ()))
    psum = lax.dot_general(onehot, h2, dn,
                           preferred_element_type=jnp.float32)
    cnt = lax.dot_general(onehot, jnp.ones((N, 1), jnp.float32), dn,
                          preferred_element_type=jnp.float32)
    pooled = psum / jnp.maximum(cnt, 1.0)
    out_ref[...] = jnp.dot(pooled, wlin_ref[...],
                           preferred_element_type=jnp.float32) + blin_ref[...]


def kernel(x, edge_index, batch, W1, b1, W2, b2, Wlin, blin):
    src4 = edge_index[0].reshape(NW, NPH, PC, CH)
    dst4 = edge_index[1].reshape(NW, NPH, PC, CH)
    dst3 = edge_index[1].reshape(NW, NCH, CH)
    ones_ch = jnp.ones((CH,), jnp.float32)
    zeros_n1 = jnp.zeros((N,), jnp.float32)
    batch2d = batch.reshape(N, 1)
    b1r = b1.reshape(1, H)
    b2r = b2.reshape(1, H)
    blinr = blin.reshape(1, C)

    degp = _sc_deg(dst3, ones_ch, zeros_n1).reshape(NC, N, 1)

    xs1, dinv = pl.pallas_call(
        _tc1_body,
        out_shape=[jax.ShapeDtypeStruct((N, H), jnp.float32),
                   jax.ShapeDtypeStruct((N, 1), jnp.float32)],
    )(x, W1, degp)

    p = _sc_edge(xs1, src4, dst4)

    xs2 = pl.pallas_call(
        _tc2_body,
        out_shape=jax.ShapeDtypeStruct((N, H), jnp.float32),
    )(p, xs1, dinv, b1r, W2)

    q = _sc_edge(xs2, src4, dst4)

    logits = pl.pallas_call(
        _tc3_body,
        out_shape=jax.ShapeDtypeStruct((G, C), jnp.float32),
    )(q, xs2, dinv, b2r, batch2d, Wlin, blinr)
    return logits


# speedup vs baseline: 32.4322x; 1.1230x over previous
import jax
import jax.numpy as jnp
from jax import lax
from jax.experimental import pallas as pl
from jax.experimental.pallas import tpu as pltpu
from jax.experimental.pallas import tpu_sc as plsc

N = 10000
E = 320000
F = 128
H = 128
C = 10
G = 128

NC = 2
NS = 16
NW = NC * NS
EP = E // NW
CH = 80
NCH = EP // CH

_SC_MESH = plsc.VectorSubcoreMesh(
    core_axis_name="c", subcore_axis_name="s", num_cores=NC, num_subcores=NS)


def _split_copy(src_ref, dst_ref, s):
    @pl.when(s < NS - 1)
    def _():
        pltpu.sync_copy(src_ref.at[pl.ds(s * 624, 624)],
                        dst_ref.at[pl.ds(s * 624, 624)])

    @pl.when(s == NS - 1)
    def _():
        pltpu.sync_copy(src_ref.at[pl.ds(9360, 640)],
                        dst_ref.at[pl.ds(9360, 640)])


def _sc_deg_body(dst_hbm, ones_hbm, zeros_hbm, out_hbm, dst_v, ones_v, acc):
    c = lax.axis_index("c")
    s = lax.axis_index("s")
    w = c * NS + s
    pltpu.sync_copy(dst_hbm.at[w], dst_v)
    pltpu.sync_copy(ones_hbm, ones_v)

    @pl.when(s == 0)
    def _():
        pltpu.sync_copy(zeros_hbm, acc)
    plsc.subcore_barrier()

    def step(j, carry):
        pltpu.sync_copy(ones_v, acc.at[dst_v.at[j]], add=True)
        return carry

    lax.fori_loop(0, NCH, step, 0)
    plsc.subcore_barrier()

    @pl.when(s == 0)
    def _():
        pltpu.sync_copy(acc, out_hbm.at[c])


_sc_deg = pl.kernel(
    _sc_deg_body,
    out_type=jax.ShapeDtypeStruct((NC, N), jnp.float32),
    mesh=_SC_MESH,
    scratch_types=[
        pltpu.VMEM((NCH, CH), jnp.int32),
        pltpu.VMEM((CH,), jnp.float32),
        pltpu.VMEM_SHARED((N,), jnp.float32),
    ],
)


NPH = 5
PC = NCH // NPH


def _sc_edge_body(xs_hbm, src_hbm, dst_hbm, out_hbm,
                  src_v, dst_v, rows, sem_g, sem_s, acc):
    c = lax.axis_index("c")
    s = lax.axis_index("s")
    w = c * NS + s
    _split_copy(xs_hbm, acc, s)
    plsc.subcore_barrier()

    def phase(p, carry):
        pltpu.sync_copy(src_hbm.at[w, p], src_v)
        pltpu.sync_copy(dst_hbm.at[w, p], dst_v)

        def virt(j, b):
            @pl.when(jnp.logical_and(j >= 3, j < PC))
            def _():
                pltpu.make_async_copy(
                    rows[b], acc.at[dst_v.at[0]], sem_s[b]).wait()

            @pl.when(j < PC)
            def _():
                pltpu.async_copy(xs_hbm.at[src_v.at[j]], rows[b], sem_g[b])

            @pl.when(j >= 2)
            def _():
                jj = j - 2
                bb = (b + 1) % 3
                pltpu.make_async_copy(
                    xs_hbm.at[src_v.at[jj]], rows[bb], sem_g[bb]).wait()
                pltpu.async_copy(rows[bb], acc.at[dst_v.at[jj]], sem_s[bb],
                                 add=True)

        def step(m, carry2):
            for i in range(3):
                virt(3 * m + i, i)
            return carry2

        lax.fori_loop(0, (PC + 2) // 3, step, 0)
        for b in range(3):
            pltpu.make_async_copy(rows[b], acc.at[dst_v.at[0]], sem_s[b]).wait()
        return carry

    lax.fori_loop(0, NPH, phase, 0)
    plsc.subcore_barrier()
    _split_copy(acc, out_hbm.at[c], s)


def _sc_edge_entry(xs_hbm, src_hbm, dst_hbm, out_hbm, src_v, dst_v,
                   rows0, rows1, rows2, sg0, sg1, sg2, ss0, ss1, ss2, acc):
    return _sc_edge_body(xs_hbm, src_hbm, dst_hbm, out_hbm, src_v, dst_v,
                         [rows0, rows1, rows2], [sg0, sg1, sg2],
                         [ss0, ss1, ss2], acc)


_sc_edge = pl.kernel(
    _sc_edge_entry,
    out_type=jax.ShapeDtypeStruct((NC, N, H), jnp.float32),
    mesh=_SC_MESH,
    scratch_types=[
        pltpu.VMEM((PC, CH), jnp.int32),
        pltpu.VMEM((PC, CH), jnp.int32),
        pltpu.VMEM((CH, H), jnp.float32),
        pltpu.VMEM((CH, H), jnp.float32),
        pltpu.VMEM((CH, H), jnp.float32),
        pltpu.SemaphoreType.DMA,
        pltpu.SemaphoreType.DMA,
        pltpu.SemaphoreType.DMA,
        pltpu.SemaphoreType.DMA,
        pltpu.SemaphoreType.DMA,
        pltpu.SemaphoreType.DMA,
        pltpu.VMEM_SHARED((N, H), jnp.float32),
    ],
)


def _tc1_body(x_ref, w1_ref, degp_ref, xs_ref, dinv_ref):
    dp = degp_ref[...]
    dinv = lax.rsqrt(dp[0] + dp[1] + 1.0)
    xw = jnp.dot(x_ref[...], w1_ref[...], preferred_element_type=jnp.float32)
    xs_ref[...] = xw * dinv
    dinv_ref[...] = dinv


def _tc2_body(p_ref, xs1_ref, dinv_ref, b1_ref, w2_ref, xs2_ref):
    p = p_ref[...]
    dinv = dinv_ref[...]
    h1 = jnp.maximum(dinv * (p[0] + p[1] - xs1_ref[...]) + b1_ref[...], 0.0)
    xs2_ref[...] = dinv * jnp.dot(h1, w2_ref[...],
                                  preferred_element_type=jnp.float32)


def _tc3_body(q_ref, xs2_ref, dinv_ref, b2_ref, batch_ref, wlin_ref, blin_ref,
              out_ref):
    q = q_ref[...]
    dinv = dinv_ref[...]
    h2 = jnp.maximum(dinv * (q[0] + q[1] - xs2_ref[...]) + b2_ref[...], 0.0)
    gids = lax.broadcasted_iota(jnp.int32, (N, G), 1)
    onehot = (batch_ref[...] == gids).astype(jnp.float32)
    dn = (((0,), (0,)), ((), ()))
    psum = lax.dot_general(onehot, h2, dn,
                           preferred_element_type=jnp.float32)
    cnt = lax.dot_general(onehot, jnp.ones((N, 1), jnp.float32), dn,
                          preferred_element_type=jnp.float32)
    pooled = psum / jnp.maximum(cnt, 1.0)
    out_ref[...] = jnp.dot(pooled, wlin_ref[...],
                           preferred_element_type=jnp.float32) + blin_ref[...]


def kernel(x, edge_index, batch, W1, b1, W2, b2, Wlin, blin):
    src4 = edge_index[0].reshape(NW, NPH, PC, CH)
    dst4 = edge_index[1].reshape(NW, NPH, PC, CH)
    dst3 = edge_index[1].reshape(NW, NCH, CH)
    ones_ch = jnp.ones((CH,), jnp.float32)
    zeros_n1 = jnp.zeros((N,), jnp.float32)
    batch2d = batch.reshape(N, 1)
    b1r = b1.reshape(1, H)
    b2r = b2.reshape(1, H)
    blinr = blin.reshape(1, C)

    degp = _sc_deg(dst3, ones_ch, zeros_n1).reshape(NC, N, 1)

    xs1, dinv = pl.pallas_call(
        _tc1_body,
        out_shape=[jax.ShapeDtypeStruct((N, H), jnp.float32),
                   jax.ShapeDtypeStruct((N, 1), jnp.float32)],
    )(x, W1, degp)

    p = _sc_edge(xs1, src4, dst4)

    xs2 = pl.pallas_call(
        _tc2_body,
        out_shape=jax.ShapeDtypeStruct((N, H), jnp.float32),
    )(p, xs1, dinv, b1r, W2)

    q = _sc_edge(xs2, src4, dst4)

    logits = pl.pallas_call(
        _tc3_body,
        out_shape=jax.ShapeDtypeStruct((G, C), jnp.float32),
    )(q, xs2, dinv, b2r, batch2d, Wlin, blinr)
    return logits


# speedup vs baseline: 32.6937x; 1.0081x over previous
import jax
import jax.numpy as jnp
from jax import lax
from jax.experimental import pallas as pl
from jax.experimental.pallas import tpu as pltpu
from jax.experimental.pallas import tpu_sc as plsc

N = 10000
E = 320000
F = 128
H = 128
C = 10
G = 128

NC = 2
NS = 16
NW = NC * NS
EP = E // NW
CH = 80
NCH = EP // CH

_SC_MESH = plsc.VectorSubcoreMesh(
    core_axis_name="c", subcore_axis_name="s", num_cores=NC, num_subcores=NS)


def _split_copy(src_ref, dst_ref, s):
    @pl.when(s < NS - 1)
    def _():
        pltpu.sync_copy(src_ref.at[pl.ds(s * 624, 624)],
                        dst_ref.at[pl.ds(s * 624, 624)])

    @pl.when(s == NS - 1)
    def _():
        pltpu.sync_copy(src_ref.at[pl.ds(9360, 640)],
                        dst_ref.at[pl.ds(9360, 640)])


def _sc_deg_body(dst_hbm, ones_hbm, zeros_hbm, out_hbm, dst_v, ones_v, acc):
    c = lax.axis_index("c")
    s = lax.axis_index("s")
    w = c * NS + s
    pltpu.sync_copy(dst_hbm.at[w], dst_v)
    pltpu.sync_copy(ones_hbm, ones_v)

    @pl.when(s == 0)
    def _():
        pltpu.sync_copy(zeros_hbm, acc)
    plsc.subcore_barrier()

    def step(j, carry):
        pltpu.sync_copy(ones_v, acc.at[dst_v.at[j]], add=True)
        return carry

    lax.fori_loop(0, NCH, step, 0)
    plsc.subcore_barrier()

    @pl.when(s == 0)
    def _():
        pltpu.sync_copy(acc, out_hbm.at[c])


_sc_deg = pl.kernel(
    _sc_deg_body,
    out_type=jax.ShapeDtypeStruct((NC, N), jnp.float32),
    mesh=_SC_MESH,
    scratch_types=[
        pltpu.VMEM((NCH, CH), jnp.int32),
        pltpu.VMEM((CH,), jnp.float32),
        pltpu.VMEM_SHARED((N,), jnp.float32),
    ],
)


NPH = 5
PC = NCH // NPH


def _sc_edge_body(xs_hbm, src_hbm, dst_hbm, out_hbm,
                  src_v, dst_v, rows, sem_g, sem_s, acc):
    c = lax.axis_index("c")
    s = lax.axis_index("s")
    w = c * NS + s
    _split_copy(xs_hbm, acc, s)
    plsc.subcore_barrier()

    def phase(p, carry):
        pltpu.sync_copy(src_hbm.at[w, p], src_v)
        pltpu.sync_copy(dst_hbm.at[w, p], dst_v)

        def virt(j, b):
            @pl.when(jnp.logical_and(j >= DEPTH, j < PC))
            def _():
                pltpu.make_async_copy(
                    rows[b], acc.at[dst_v.at[0]], sem_s[b]).wait()

            @pl.when(j < PC)
            def _():
                pltpu.async_copy(xs_hbm.at[src_v.at[j]], rows[b], sem_g[b])

            @pl.when(j >= DEPTH - 1)
            def _():
                jj = j - (DEPTH - 1)
                bb = (b + 1) % DEPTH
                pltpu.make_async_copy(
                    xs_hbm.at[src_v.at[jj]], rows[bb], sem_g[bb]).wait()
                pltpu.async_copy(rows[bb], acc.at[dst_v.at[jj]], sem_s[bb],
                                 add=True)

        def step(m, carry2):
            for i in range(DEPTH):
                virt(DEPTH * m + i, i)
            return carry2

        lax.fori_loop(0, (PC + DEPTH - 1) // DEPTH, step, 0)
        for b in range(DEPTH):
            pltpu.make_async_copy(rows[b], acc.at[dst_v.at[0]], sem_s[b]).wait()
        return carry

    lax.fori_loop(0, NPH, phase, 0)
    plsc.subcore_barrier()
    _split_copy(acc, out_hbm.at[c], s)


DEPTH = 4
assert (PC + DEPTH - 1) % DEPTH == 0


def _sc_edge_entry(xs_hbm, src_hbm, dst_hbm, out_hbm, src_v, dst_v,
                   rows_b, sem_g_b, sem_s_b, acc):
    return _sc_edge_body(xs_hbm, src_hbm, dst_hbm, out_hbm, src_v, dst_v,
                         list(rows_b), list(sem_g_b), list(sem_s_b), acc)


_sc_edge = pl.kernel(
    _sc_edge_entry,
    out_type=jax.ShapeDtypeStruct((NC, N, H), jnp.float32),
    mesh=_SC_MESH,
    scratch_types=[
        pltpu.VMEM((PC, CH), jnp.int32),
        pltpu.VMEM((PC, CH), jnp.int32),
        [pltpu.VMEM((CH, H), jnp.float32) for _ in range(DEPTH)],
        [pltpu.SemaphoreType.DMA for _ in range(DEPTH)],
        [pltpu.SemaphoreType.DMA for _ in range(DEPTH)],
        pltpu.VMEM_SHARED((N, H), jnp.float32),
    ],
)


def _tc1_body(x_ref, w1_ref, degp_ref, xs_ref, dinv_ref):
    dp = degp_ref[...]
    dinv = lax.rsqrt(dp[0] + dp[1] + 1.0)
    xw = jnp.dot(x_ref[...], w1_ref[...], preferred_element_type=jnp.float32)
    xs_ref[...] = xw * dinv
    dinv_ref[...] = dinv


def _tc2_body(p_ref, xs1_ref, dinv_ref, b1_ref, w2_ref, xs2_ref):
    p = p_ref[...]
    dinv = dinv_ref[...]
    h1 = jnp.maximum(dinv * (p[0] + p[1] - xs1_ref[...]) + b1_ref[...], 0.0)
    xs2_ref[...] = dinv * jnp.dot(h1, w2_ref[...],
                                  preferred_element_type=jnp.float32)


def _tc3_body(q_ref, xs2_ref, dinv_ref, b2_ref, batch_ref, wlin_ref, blin_ref,
              out_ref):
    q = q_ref[...]
    dinv = dinv_ref[...]
    h2 = jnp.maximum(dinv * (q[0] + q[1] - xs2_ref[...]) + b2_ref[...], 0.0)
    gids = lax.broadcasted_iota(jnp.int32, (N, G), 1)
    onehot = (batch_ref[...] == gids).astype(jnp.float32)
    dn = (((0,), (0,)), ((), ()))
    psum = lax.dot_general(onehot, h2, dn,
                           preferred_element_type=jnp.float32)
    cnt = lax.dot_general(onehot, jnp.ones((N, 1), jnp.float32), dn,
                          preferred_element_type=jnp.float32)
    pooled = psum / jnp.maximum(cnt, 1.0)
    out_ref[...] = jnp.dot(pooled, wlin_ref[...],
                           preferred_element_type=jnp.float32) + blin_ref[...]


def kernel(x, edge_index, batch, W1, b1, W2, b2, Wlin, blin):
    src4 = edge_index[0].reshape(NW, NPH, PC, CH)
    dst4 = edge_index[1].reshape(NW, NPH, PC, CH)
    dst3 = edge_index[1].reshape(NW, NCH, CH)
    ones_ch = jnp.ones((CH,), jnp.float32)
    zeros_n1 = jnp.zeros((N,), jnp.float32)
    batch2d = batch.reshape(N, 1)
    b1r = b1.reshape(1, H)
    b2r = b2.reshape(1, H)
    blinr = blin.reshape(1, C)

    degp = _sc_deg(dst3, ones_ch, zeros_n1).reshape(NC, N, 1)

    xs1, dinv = pl.pallas_call(
        _tc1_body,
        out_shape=[jax.ShapeDtypeStruct((N, H), jnp.float32),
                   jax.ShapeDtypeStruct((N, 1), jnp.float32)],
    )(x, W1, degp)

    p = _sc_edge(xs1, src4, dst4)

    xs2 = pl.pallas_call(
        _tc2_body,
        out_shape=jax.ShapeDtypeStruct((N, H), jnp.float32),
    )(p, xs1, dinv, b1r, W2)

    q = _sc_edge(xs2, src4, dst4)

    logits = pl.pallas_call(
        _tc3_body,
        out_shape=jax.ShapeDtypeStruct((G, C), jnp.float32),
    )(q, xs2, dinv, b2r, batch2d, Wlin, blinr)
    return logits


# speedup vs baseline: 33.4965x; 1.0246x over previous
import jax
import jax.numpy as jnp
from jax import lax
from jax.experimental import pallas as pl
from jax.experimental.pallas import tpu as pltpu
from jax.experimental.pallas import tpu_sc as plsc

N = 10000
E = 320000
F = 128
H = 128
C = 10
G = 128

NC = 2
NS = 16
NW = NC * NS
EP = E // NW
CH = 80
NCH = EP // CH

_SC_MESH = plsc.VectorSubcoreMesh(
    core_axis_name="c", subcore_axis_name="s", num_cores=NC, num_subcores=NS)


def _split_copy(src_ref, dst_ref, s):
    @pl.when(s < NS - 1)
    def _():
        pltpu.sync_copy(src_ref.at[pl.ds(s * 624, 624)],
                        dst_ref.at[pl.ds(s * 624, 624)])

    @pl.when(s == NS - 1)
    def _():
        pltpu.sync_copy(src_ref.at[pl.ds(9360, 640)],
                        dst_ref.at[pl.ds(9360, 640)])


def _sc_deg_body(dst_hbm, ones_hbm, zeros_hbm, out_hbm, dst_v, ones_v, acc,
                 sem_d):
    c = lax.axis_index("c")
    s = lax.axis_index("s")
    w = c * NS + s
    pltpu.sync_copy(dst_hbm.at[w], dst_v)
    pltpu.sync_copy(ones_hbm, ones_v)

    @pl.when(s == 0)
    def _():
        pltpu.sync_copy(zeros_hbm, acc)
    plsc.subcore_barrier()

    def step(k, carry):
        for i in range(5):
            pltpu.async_copy(ones_v, acc.at[dst_v.at[5 * k + i]], sem_d,
                             add=True)
        for i in range(5):
            pltpu.make_async_copy(ones_v, acc.at[dst_v.at[0]], sem_d).wait()
        return carry

    lax.fori_loop(0, NCH // 5, step, 0)
    plsc.subcore_barrier()

    @pl.when(s == 0)
    def _():
        pltpu.sync_copy(acc, out_hbm.at[c])


_sc_deg = pl.kernel(
    _sc_deg_body,
    out_type=jax.ShapeDtypeStruct((NC, N), jnp.float32),
    mesh=_SC_MESH,
    scratch_types=[
        pltpu.VMEM((NCH, CH), jnp.int32),
        pltpu.VMEM((CH,), jnp.float32),
        pltpu.VMEM_SHARED((N,), jnp.float32),
        pltpu.SemaphoreType.DMA,
    ],
)


NPH = 5
PC = NCH // NPH


def _sc_edge_body(xs_hbm, src_hbm, dst_hbm, zer_hbm, out_hbm,
                  src_v, dst_v, rows, sem_g, sem_s, acc):
    c = lax.axis_index("c")
    s = lax.axis_index("s")
    w = c * NS + s
    @pl.when(c == 0)
    def _():
        _split_copy(xs_hbm, acc, s)

    @pl.when(c == 1)
    def _():
        _split_copy(zer_hbm, acc, s)
    plsc.subcore_barrier()

    def phase(p, carry):
        pltpu.sync_copy(src_hbm.at[w, p], src_v)
        pltpu.sync_copy(dst_hbm.at[w, p], dst_v)

        def virt(j, b):
            @pl.when(jnp.logical_and(j >= DEPTH, j < PC))
            def _():
                pltpu.make_async_copy(
                    rows[b], acc.at[dst_v.at[0]], sem_s[b]).wait()

            @pl.when(j < PC)
            def _():
                pltpu.async_copy(xs_hbm.at[src_v.at[j]], rows[b], sem_g[b])

            @pl.when(j >= DEPTH - 1)
            def _():
                jj = j - (DEPTH - 1)
                bb = (b + 1) % DEPTH
                pltpu.make_async_copy(
                    xs_hbm.at[src_v.at[jj]], rows[bb], sem_g[bb]).wait()
                pltpu.async_copy(rows[bb], acc.at[dst_v.at[jj]], sem_s[bb],
                                 add=True)

        def step(m, carry2):
            for i in range(DEPTH):
                virt(DEPTH * m + i, i)
            return carry2

        lax.fori_loop(0, (PC + DEPTH - 1) // DEPTH, step, 0)
        for b in range(DEPTH):
            pltpu.make_async_copy(rows[b], acc.at[dst_v.at[0]], sem_s[b]).wait()
        return carry

    lax.fori_loop(0, NPH, phase, 0)
    plsc.subcore_barrier()
    _split_copy(acc, out_hbm.at[c], s)


DEPTH = 4
assert (PC + DEPTH - 1) % DEPTH == 0


def _sc_edge_entry(xs_hbm, src_hbm, dst_hbm, zer_hbm, out_hbm, src_v, dst_v,
                   rows_b, sem_g_b, sem_s_b, acc):
    return _sc_edge_body(xs_hbm, src_hbm, dst_hbm, zer_hbm, out_hbm,
                         src_v, dst_v,
                         list(rows_b), list(sem_g_b), list(sem_s_b), acc)


_sc_edge = pl.kernel(
    _sc_edge_entry,
    out_type=jax.ShapeDtypeStruct((NC, N, H), jnp.float32),
    mesh=_SC_MESH,
    scratch_types=[
        pltpu.VMEM((PC, CH), jnp.int32),
        pltpu.VMEM((PC, CH), jnp.int32),
        [pltpu.VMEM((CH, H), jnp.float32) for _ in range(DEPTH)],
        [pltpu.SemaphoreType.DMA for _ in range(DEPTH)],
        [pltpu.SemaphoreType.DMA for _ in range(DEPTH)],
        pltpu.VMEM_SHARED((N, H), jnp.float32),
    ],
)


def _tc1_body(x_ref, w1_ref, degp_ref, xs_ref, dinv_ref):
    dp = degp_ref[...]
    dinv = lax.rsqrt(dp[0] + dp[1] + 1.0)
    xw = jnp.dot(x_ref[...], w1_ref[...], preferred_element_type=jnp.float32)
    xs_ref[...] = xw * dinv
    dinv_ref[...] = dinv


def _tc2_body(p_ref, dinv_ref, b1_ref, w2_ref, xs2_ref):
    p = p_ref[...]
    dinv = dinv_ref[...]
    h1 = jnp.maximum(dinv * (p[0] + p[1]) + b1_ref[...], 0.0)
    xs2_ref[...] = dinv * jnp.dot(h1, w2_ref[...],
                                  preferred_element_type=jnp.float32)


def _tc3_body(q_ref, dinv_ref, b2_ref, batch_ref, wlin_ref, blin_ref,
              out_ref):
    q = q_ref[...]
    dinv = dinv_ref[...]
    h2 = jnp.maximum(dinv * (q[0] + q[1]) + b2_ref[...], 0.0)
    gids = lax.broadcasted_iota(jnp.int32, (N, G), 1)
    onehot = (batch_ref[...] == gids).astype(jnp.float32)
    dn = (((0,), (0,)), ((), ()))
    psum = lax.dot_general(onehot, h2, dn,
                           preferred_element_type=jnp.float32)
    cnt = lax.dot_general(onehot, jnp.ones((N, 1), jnp.float32), dn,
                          preferred_element_type=jnp.float32)
    pooled = psum / jnp.maximum(cnt, 1.0)
    out_ref[...] = jnp.dot(pooled, wlin_ref[...],
                           preferred_element_type=jnp.float32) + blin_ref[...]


def kernel(x, edge_index, batch, W1, b1, W2, b2, Wlin, blin):
    src4 = edge_index[0].reshape(NW, NPH, PC, CH)
    dst4 = edge_index[1].reshape(NW, NPH, PC, CH)
    dst3 = edge_index[1].reshape(NW, NCH, CH)
    ones_ch = jnp.ones((CH,), jnp.float32)
    zeros_n1 = jnp.zeros((N,), jnp.float32)
    zeros_nf = jnp.zeros((N, H), jnp.float32)
    batch2d = batch.reshape(N, 1)
    b1r = b1.reshape(1, H)
    b2r = b2.reshape(1, H)
    blinr = blin.reshape(1, C)

    degp = _sc_deg(dst3, ones_ch, zeros_n1).reshape(NC, N, 1)

    xs1, dinv = pl.pallas_call(
        _tc1_body,
        out_shape=[jax.ShapeDtypeStruct((N, H), jnp.float32),
                   jax.ShapeDtypeStruct((N, 1), jnp.float32)],
    )(x, W1, degp)

    p = _sc_edge(xs1, src4, dst4, zeros_nf)

    xs2 = pl.pallas_call(
        _tc2_body,
        out_shape=jax.ShapeDtypeStruct((N, H), jnp.float32),
    )(p, dinv, b1r, W2)

    q = _sc_edge(xs2, src4, dst4, zeros_nf)

    logits = pl.pallas_call(
        _tc3_body,
        out_shape=jax.ShapeDtypeStruct((G, C), jnp.float32),
    )(q, dinv, b2r, batch2d, Wlin, blinr)
    return logits
